# Initial kernel scaffold; baseline (speedup 1.0000x reference)
import jax
import jax.numpy as jnp
from jax.experimental import pallas as pl


def kernel(H_t, X_t, cond_embedding, edges, edge_types, generate_mask, batch_ids, t, params):
    raise NotImplementedError("write your pallas kernel here")


# speedup vs baseline: 2.1883x; 2.1883x over previous
import functools

import jax
import jax.numpy as jnp
from jax import lax
from jax.experimental import pallas as pl
from jax.experimental.pallas import tpu as pltpu
from jax.experimental.pallas import tpu_sc as plsc

NN = 10000
EE = 160000
HID = 128
GW = 256
PW = 128
NC, NS = 2, 16
NWORK = NC * NS
E_PAD = 163840
EPW = E_PAD // NWORK
CH = 128
NCHUNK = EPW // CH
N_ACC = 10112
RPS = N_ACC // NS
BL = 1000
NB = NN // BL
EBL = 1024
NEB = E_PAD // EBL

_f32 = jnp.float32


def _wspec(shape):
    return pl.BlockSpec(shape, lambda *_: tuple(0 for _ in shape))


def _prelude_body(ht, cond, tcol, w0a, w0b, w0c, b0, w1, b1, w2, b2, out):
    half = HID // 2
    i = lax.broadcasted_iota(jnp.int32, (1, half), 1).astype(_f32)
    freqs = jnp.exp(-jnp.log(10000.0) * i / (half - 1))
    ang = tcol[...] * freqs
    temb = jnp.concatenate([jnp.sin(ang), jnp.cos(ang)], axis=1)
    h = ht[...] @ w0a[...] + cond[...] @ w0b[...] + temb @ w0c[...] + b0[...]
    h = jax.nn.relu(h)
    h = jax.nn.relu(h @ w1[...] + b1[...])
    out[...] = h @ w2[...] + b2[...]


def _prelude(H_t, cond, tcol, p0, p1, p2):
    w0 = p0["w"]
    return pl.pallas_call(
        _prelude_body,
        grid=(NB,),
        in_specs=[
            pl.BlockSpec((BL, HID), lambda i: (i, 0)),
            pl.BlockSpec((BL, HID), lambda i: (i, 0)),
            pl.BlockSpec((BL, 1), lambda i: (i, 0)),
            _wspec((HID, HID)), _wspec((HID, HID)), _wspec((HID, HID)),
            _wspec((1, HID)), _wspec((HID, HID)), _wspec((1, HID)),
            _wspec((HID, HID)), _wspec((1, HID)),
        ],
        out_specs=pl.BlockSpec((BL, HID), lambda i: (i, 0)),
        out_shape=jax.ShapeDtypeStruct((NN, HID), _f32),
    )(H_t, cond, tcol, w0[:HID], w0[HID:2 * HID], w0[2 * HID:],
      p0["b"][None, :], p1["w"], p1["b"][None, :], p2["w"], p2["b"][None, :])


def _tables_body(h, xp, w1r, w1c, teff, ta, tb):
    a = jnp.concatenate([h[...] @ w1r[...], xp[...]], axis=1)
    ta[...] = a
    b = jnp.concatenate([h[...] @ w1c[...] + teff[0], xp[...]], axis=1)
    tb[...] = b


def _tables(h, xpad, w1r, w1c_b, teff):
    return pl.pallas_call(
        _tables_body,
        grid=(2, NB),
        in_specs=[
            pl.BlockSpec((BL, HID), lambda s, i: (i, 0)),
            pl.BlockSpec((BL, HID), lambda s, i: (i, 0)),
            _wspec((HID, HID)), _wspec((HID, HID)),
            pl.BlockSpec((1, 1, HID), lambda s, i: (s, 0, 0)),
        ],
        out_specs=[
            pl.BlockSpec((BL, GW), lambda s, i: (i, 0)),
            pl.BlockSpec((BL, GW), lambda s, i: (s * NB + i, 0)),
        ],
        out_shape=[
            jax.ShapeDtypeStruct((NN, GW), _f32),
            jax.ShapeDtypeStruct((2 * NN, GW), _f32),
        ],
    )(h, xpad, w1r, w1c_b, teff)


def _sc_gather_body(ta, tb2, rowp, colp, etp, ga, gb,
                    idxr, idxc, idxt, bufa, bufb, sema, semb):
    wid = lax.axis_index("s") * NC + lax.axis_index("c")
    base0 = wid * EPW

    def chunk(i, carry):
        base = base0 + i * CH
        pltpu.sync_copy(rowp.at[pl.ds(base, CH)], idxr)
        pltpu.sync_copy(colp.at[pl.ds(base, CH)], idxc)
        pltpu.sync_copy(etp.at[pl.ds(base, CH)], idxt)
        for j in range(CH // 16):
            sl = pl.ds(j * 16, 16)
            idxc[sl] = idxc[sl] + idxt[sl] * NN
        ca = pltpu.async_copy(ta.at[idxr], bufa, sema)
        cb = pltpu.async_copy(tb2.at[idxc], bufb, semb)
        ca.wait()
        cb.wait()
        pltpu.sync_copy(bufa, ga.at[pl.ds(base, CH)])
        pltpu.sync_copy(bufb, gb.at[pl.ds(base, CH)])
        return carry

    lax.fori_loop(0, NCHUNK, chunk, 0)


def _sc_gather(ta, tb2, rowp, colp, etp):
    mesh = plsc.VectorSubcoreMesh(core_axis_name="c", subcore_axis_name="s",
                                  num_cores=NC, num_subcores=NS)
    return pl.kernel(
        _sc_gather_body,
        out_type=[
            jax.ShapeDtypeStruct((E_PAD, GW), _f32),
            jax.ShapeDtypeStruct((E_PAD, GW), _f32),
        ],
        mesh=mesh,
        scratch_types=[
            pltpu.VMEM((CH,), jnp.int32),
            pltpu.VMEM((CH,), jnp.int32),
            pltpu.VMEM((CH,), jnp.int32),
            pltpu.VMEM((CH, GW), _f32),
            pltpu.VMEM((CH, GW), _f32),
            pltpu.SemaphoreType.DMA,
            pltpu.SemaphoreType.DMA,
        ],
    )(ta, tb2, rowp, colp, etp)


def _edge_body(ga, gb, wd2, w2, b2, wc, bc, om, oc):
    a = ga[...]
    b = gb[...]
    rel = a[:, HID:] - b[:, HID:]
    d2 = jnp.sum(rel * rel, axis=1, keepdims=True)
    z = a[:, :HID] + b[:, :HID] + d2 * wd2[...]
    m = jax.nn.silu(z)
    m = jax.nn.silu(m @ w2[...] + b2[...])
    wco = jnp.sum(m * wc[...], axis=1, keepdims=True) + bc[...]
    lane = lax.broadcasted_iota(jnp.int32, (1, PW), 1)
    one3 = jnp.where(lane == 3, 1.0, 0.0).astype(_f32)
    om[...] = m
    oc[...] = rel * wco + one3


def _edge(ga, gb, wd2, w2, b2, wc, bc):
    return pl.pallas_call(
        _edge_body,
        grid=(NEB,),
        in_specs=[
            pl.BlockSpec((EBL, GW), lambda i: (i, 0)),
            pl.BlockSpec((EBL, GW), lambda i: (i, 0)),
            _wspec((1, HID)), _wspec((HID, HID)), _wspec((1, HID)),
            _wspec((1, HID)), _wspec((1, 1)),
        ],
        out_specs=[
            pl.BlockSpec((EBL, PW), lambda i: (i, 0)),
            pl.BlockSpec((EBL, PW), lambda i: (i, 0)),
        ],
        out_shape=[
            jax.ShapeDtypeStruct((E_PAD, PW), _f32),
            jax.ShapeDtypeStruct((E_PAD, PW), _f32),
        ],
    )(ga, gb, wd2, w2, b2, wc, bc)


def _sc_scatter_body(s1, s2, colsp, zsrc, o1, o2, idxv, buf, shared, sem):
    c = lax.axis_index("c")
    s = lax.axis_index("s")
    base0 = (c * NS + s) * EPW
    rsl = pl.ds(s * RPS, RPS)

    for src, out in ((s1, o1), (s2, o2)):
        pltpu.sync_copy(zsrc.at[rsl], shared.at[rsl])
        plsc.subcore_barrier()

        def chunk(i, carry):
            base = base0 + i * CH
            pltpu.sync_copy(colsp.at[pl.ds(base, CH)], idxv)
            pltpu.sync_copy(src.at[pl.ds(base, CH)], buf)
            pltpu.sync_copy(buf, shared.at[idxv], add=True)
            return carry

        lax.fori_loop(0, NCHUNK, chunk, 0)
        plsc.subcore_barrier()
        pltpu.sync_copy(shared.at[rsl], out.at[c, rsl])
        plsc.subcore_barrier()


def _sc_scatter(s1, s2, colsp, zsrc):
    mesh = plsc.VectorSubcoreMesh(core_axis_name="c", subcore_axis_name="s",
                                  num_cores=NC, num_subcores=NS)
    return pl.kernel(
        _sc_scatter_body,
        out_type=[
            jax.ShapeDtypeStruct((NC, N_ACC, PW), _f32),
            jax.ShapeDtypeStruct((NC, N_ACC, PW), _f32),
        ],
        mesh=mesh,
        scratch_types=[
            pltpu.VMEM((CH,), jnp.int32),
            pltpu.VMEM((CH, PW), _f32),
            pltpu.VMEM_SHARED((N_ACC, PW), _f32),
            pltpu.SemaphoreType.DMA,
        ],
    )(s1, s2, colsp, zsrc)


def _update_body(m0, m1, c0, c1, h, xp, u1a, u1b, bu1, u2, b2u, hn, xn):
    aggm = m0[0] + m1[0]
    aggc = c0[0] + c1[0]
    lane = lax.broadcasted_iota(jnp.int32, (1, PW), 1)
    mask3 = jnp.where(lane < 3, 1.0, 0.0).astype(_f32)
    deg = aggc[:, 3:4]
    xn[...] = xp[...] + (aggc * mask3) / (deg + 1.0)
    u = jax.nn.silu(h[...] @ u1a[...] + aggm @ u1b[...] + bu1[...])
    hn[...] = h[...] + u @ u2[...] + b2u[...]


def _update(aggm2, aggc2, h, xpad, u1a, u1b, bu1, u2, b2u):
    return pl.pallas_call(
        _update_body,
        grid=(NB,),
        in_specs=[
            pl.BlockSpec((1, BL, PW), lambda i: (0, i, 0)),
            pl.BlockSpec((1, BL, PW), lambda i: (1, i, 0)),
            pl.BlockSpec((1, BL, PW), lambda i: (0, i, 0)),
            pl.BlockSpec((1, BL, PW), lambda i: (1, i, 0)),
            pl.BlockSpec((BL, HID), lambda i: (i, 0)),
            pl.BlockSpec((BL, PW), lambda i: (i, 0)),
            _wspec((HID, HID)), _wspec((HID, HID)), _wspec((1, HID)),
            _wspec((HID, HID)), _wspec((1, HID)),
        ],
        out_specs=[
            pl.BlockSpec((BL, HID), lambda i: (i, 0)),
            pl.BlockSpec((BL, PW), lambda i: (i, 0)),
        ],
        out_shape=[
            jax.ShapeDtypeStruct((NN, HID), _f32),
            jax.ShapeDtypeStruct((NN, PW), _f32),
        ],
    )(aggm2, aggm2, aggc2, aggc2, h, xpad, u1a, u1b, bu1, u2, b2u)


def _final_body(h, xp, mk, wh, bh, vh, vx):
    vh[...] = (h[...] @ wh[...] + bh[...]) * mk[...]
    vx[...] = xp[...] * mk[...]


def _final(h, xpad, maskcol, wh, bh):
    return pl.pallas_call(
        _final_body,
        grid=(NB,),
        in_specs=[
            pl.BlockSpec((BL, HID), lambda i: (i, 0)),
            pl.BlockSpec((BL, PW), lambda i: (i, 0)),
            pl.BlockSpec((BL, 1), lambda i: (i, 0)),
            _wspec((HID, HID)), _wspec((1, HID)),
        ],
        out_specs=[
            pl.BlockSpec((BL, HID), lambda i: (i, 0)),
            pl.BlockSpec((BL, PW), lambda i: (i, 0)),
        ],
        out_shape=[
            jax.ShapeDtypeStruct((NN, HID), _f32),
            jax.ShapeDtypeStruct((NN, PW), _f32),
        ],
    )(h, xpad, maskcol, wh, bh)


@jax.jit
def _run(H_t, X_t, cond_embedding, edges, edge_types, generate_mask, t, params):
    row = edges[0].astype(jnp.int32)
    col = edges[1].astype(jnp.int32)
    et = edge_types.astype(jnp.int32)
    padz = jnp.zeros((E_PAD - EE,), jnp.int32)
    rowp = jnp.concatenate([row, padz])
    colp = jnp.concatenate([col, padz])
    etp = jnp.concatenate([et, padz])
    colsp = jnp.concatenate([col, jnp.full((E_PAD - EE,), NN, jnp.int32)])
    xpad = jnp.pad(X_t, ((0, 0), (0, PW - 3)))
    tcol = t[:, None]
    maskcol = generate_mask.astype(_f32)[:, None]
    zsrc = jnp.zeros((N_ACC, PW), _f32)

    p0, p1, p2 = params["input_mlp"]
    h = _prelude(H_t, cond_embedding, tcol, p0, p1, p2)

    for lp in params["layers"]:
        w1 = lp["msg1"]["w"]
        b1 = lp["msg1"]["b"]
        teff = (params["edge_table"] @ w1[2 * HID + 1:] + b1[None, :])[:, None, :]
        wd2 = w1[2 * HID:2 * HID + 1]
        ta, tb2 = _tables(h, xpad, w1[:HID], w1[HID:2 * HID], teff)
        ga, gb = _sc_gather(ta, tb2, rowp, colp, etp)
        s1, s2 = _edge(ga, gb, wd2, lp["msg2"]["w"], lp["msg2"]["b"][None, :],
                       lp["coord"]["w"].T, lp["coord"]["b"][None, None, 0])
        aggm2, aggc2 = _sc_scatter(s1, s2, colsp, zsrc)
        u1 = lp["upd1"]["w"]
        h, xpad = _update(aggm2, aggc2, h, xpad, u1[:HID], u1[HID:],
                          lp["upd1"]["b"][None, :], lp["upd2"]["w"],
                          lp["upd2"]["b"][None, :])

    vh, vxp = _final(h, xpad, maskcol, params["hidden2input"]["w"],
                     params["hidden2input"]["b"][None, :])
    return vh, vxp[:, :3]


def kernel(H_t, X_t, cond_embedding, edges, edge_types, generate_mask,
           batch_ids, t, params):
    return _run(H_t, X_t, cond_embedding, edges, edge_types, generate_mask,
                t, params)


# speedup vs baseline: 2.8918x; 1.3215x over previous
import functools

import jax
import jax.numpy as jnp
from jax import lax
from jax.experimental import pallas as pl
from jax.experimental.pallas import tpu as pltpu
from jax.experimental.pallas import tpu_sc as plsc

NN = 10000
EE = 160000
HID = 128
GW = 256
PW = 128
NC, NS = 2, 16
NWORK = NC * NS
E_PAD = 163840
EPW = E_PAD // NWORK
CH = 128
NCHUNK = EPW // CH
N_ACC = 10112
RPS = N_ACC // NS
BL = 1000
NB = NN // BL
EBL = 1024
NEB = E_PAD // EBL

_f32 = jnp.float32


def _wspec(shape):
    return pl.BlockSpec(shape, lambda *_: tuple(0 for _ in shape))


def _prelude_body(ht, cond, tcol, w0a, w0b, w0c, b0, w1, b1, w2, b2, out):
    half = HID // 2
    i = lax.broadcasted_iota(jnp.int32, (1, half), 1).astype(_f32)
    freqs = jnp.exp(-jnp.log(10000.0) * i / (half - 1))
    ang = tcol[...] * freqs
    temb = jnp.concatenate([jnp.sin(ang), jnp.cos(ang)], axis=1)
    h = ht[...] @ w0a[...] + cond[...] @ w0b[...] + temb @ w0c[...] + b0[...]
    h = jax.nn.relu(h)
    h = jax.nn.relu(h @ w1[...] + b1[...])
    out[...] = h @ w2[...] + b2[...]


def _prelude(H_t, cond, tcol, p0, p1, p2):
    w0 = p0["w"]
    return pl.pallas_call(
        _prelude_body,
        grid=(NB,),
        in_specs=[
            pl.BlockSpec((BL, HID), lambda i: (i, 0)),
            pl.BlockSpec((BL, HID), lambda i: (i, 0)),
            pl.BlockSpec((BL, 1), lambda i: (i, 0)),
            _wspec((HID, HID)), _wspec((HID, HID)), _wspec((HID, HID)),
            _wspec((1, HID)), _wspec((HID, HID)), _wspec((1, HID)),
            _wspec((HID, HID)), _wspec((1, HID)),
        ],
        out_specs=pl.BlockSpec((BL, HID), lambda i: (i, 0)),
        out_shape=jax.ShapeDtypeStruct((NN, HID), _f32),
    )(H_t, cond, tcol, w0[:HID], w0[HID:2 * HID], w0[2 * HID:],
      p0["b"][None, :], p1["w"], p1["b"][None, :], p2["w"], p2["b"][None, :])


def _tables_body(h, xp, w1r, w1c, teff, ta, tb):
    a = jnp.concatenate([h[...] @ w1r[...], xp[...]], axis=1)
    ta[...] = a
    b = jnp.concatenate([h[...] @ w1c[...] + teff[0], xp[...]], axis=1)
    tb[...] = b


def _tables(h, xpad, w1r, w1c_b, teff):
    return pl.pallas_call(
        _tables_body,
        grid=(2, NB),
        in_specs=[
            pl.BlockSpec((BL, HID), lambda s, i: (i, 0)),
            pl.BlockSpec((BL, HID), lambda s, i: (i, 0)),
            _wspec((HID, HID)), _wspec((HID, HID)),
            pl.BlockSpec((1, 1, HID), lambda s, i: (s, 0, 0)),
        ],
        out_specs=[
            pl.BlockSpec((BL, GW), lambda s, i: (i, 0)),
            pl.BlockSpec((BL, GW), lambda s, i: (s * NB + i, 0)),
        ],
        out_shape=[
            jax.ShapeDtypeStruct((NN, GW), _f32),
            jax.ShapeDtypeStruct((2 * NN, GW), _f32),
        ],
    )(h, xpad, w1r, w1c_b, teff)


def _sc_gather_body(ta, tb2, rowp, colp, etp, ga, gb,
                    idxr, idxc, idxt, bufa, bufb, sema, semb):
    wid = lax.axis_index("s") * NC + lax.axis_index("c")
    base0 = wid * EPW

    def chunk(i, carry):
        base = base0 + i * CH
        pltpu.sync_copy(rowp.at[pl.ds(base, CH)], idxr)
        pltpu.sync_copy(colp.at[pl.ds(base, CH)], idxc)
        pltpu.sync_copy(etp.at[pl.ds(base, CH)], idxt)
        for j in range(CH // 16):
            sl = pl.ds(j * 16, 16)
            idxc[sl] = idxc[sl] + idxt[sl] * NN
        ca = pltpu.async_copy(ta.at[idxr], bufa, sema)
        cb = pltpu.async_copy(tb2.at[idxc], bufb, semb)
        ca.wait()
        cb.wait()
        pltpu.sync_copy(bufa, ga.at[pl.ds(base, CH)])
        pltpu.sync_copy(bufb, gb.at[pl.ds(base, CH)])
        return carry

    lax.fori_loop(0, NCHUNK, chunk, 0)


def _sc_gather(ta, tb2, rowp, colp, etp):
    mesh = plsc.VectorSubcoreMesh(core_axis_name="c", subcore_axis_name="s",
                                  num_cores=NC, num_subcores=NS)
    return pl.kernel(
        _sc_gather_body,
        out_type=[
            jax.ShapeDtypeStruct((E_PAD, GW), _f32),
            jax.ShapeDtypeStruct((E_PAD, GW), _f32),
        ],
        mesh=mesh,
        scratch_types=[
            pltpu.VMEM((CH,), jnp.int32),
            pltpu.VMEM((CH,), jnp.int32),
            pltpu.VMEM((CH,), jnp.int32),
            pltpu.VMEM((CH, GW), _f32),
            pltpu.VMEM((CH, GW), _f32),
            pltpu.SemaphoreType.DMA,
            pltpu.SemaphoreType.DMA,
        ],
    )(ta, tb2, rowp, colp, etp)


def _edge_body(ga, gb, wd2, w2, b2, wc, bc, om, oc):
    a = ga[...]
    b = gb[...]
    rel = a[:, HID:] - b[:, HID:]
    d2 = jnp.sum(rel * rel, axis=1, keepdims=True)
    z = a[:, :HID] + b[:, :HID] + d2 * wd2[...]
    m = jax.nn.silu(z)
    m = jax.nn.silu(m @ w2[...] + b2[...])
    wco = jnp.sum(m * wc[...], axis=1, keepdims=True) + bc[...]
    lane = lax.broadcasted_iota(jnp.int32, (1, PW), 1)
    one3 = jnp.where(lane == 3, 1.0, 0.0).astype(_f32)
    om[...] = m
    oc[...] = rel * wco + one3


def _edge(ga, gb, wd2, w2, b2, wc, bc):
    return pl.pallas_call(
        _edge_body,
        grid=(NEB,),
        in_specs=[
            pl.BlockSpec((EBL, GW), lambda i: (i, 0)),
            pl.BlockSpec((EBL, GW), lambda i: (i, 0)),
            _wspec((1, HID)), _wspec((HID, HID)), _wspec((1, HID)),
            _wspec((1, HID)), _wspec((1, 1)),
        ],
        out_specs=[
            pl.BlockSpec((EBL, PW), lambda i: (i, 0)),
            pl.BlockSpec((EBL, PW), lambda i: (i, 0)),
        ],
        out_shape=[
            jax.ShapeDtypeStruct((E_PAD, PW), _f32),
            jax.ShapeDtypeStruct((E_PAD, PW), _f32),
        ],
    )(ga, gb, wd2, w2, b2, wc, bc)


def _sc_scatter_body(s1, s2, colsp, zsrc, o1, o2, idxv, buf, shared, sem):
    c = lax.axis_index("c")
    s = lax.axis_index("s")
    base0 = (c * NS + s) * EPW
    rsl = pl.ds(s * RPS, RPS)

    for src, out in ((s1, o1), (s2, o2)):
        pltpu.sync_copy(zsrc.at[rsl], shared.at[rsl])
        plsc.subcore_barrier()

        def chunk(i, carry):
            base = base0 + i * CH
            pltpu.sync_copy(colsp.at[pl.ds(base, CH)], idxv)
            pltpu.sync_copy(src.at[pl.ds(base, CH)], buf)
            pltpu.sync_copy(buf, shared.at[idxv], add=True)
            return carry

        lax.fori_loop(0, NCHUNK, chunk, 0)
        plsc.subcore_barrier()
        pltpu.sync_copy(shared.at[rsl], out.at[c, rsl])
        plsc.subcore_barrier()


def _sc_scatter(s1, s2, colsp, zsrc):
    mesh = plsc.VectorSubcoreMesh(core_axis_name="c", subcore_axis_name="s",
                                  num_cores=NC, num_subcores=NS)
    return pl.kernel(
        _sc_scatter_body,
        out_type=[
            jax.ShapeDtypeStruct((NC, N_ACC, PW), _f32),
            jax.ShapeDtypeStruct((NC, N_ACC, PW), _f32),
        ],
        mesh=mesh,
        scratch_types=[
            pltpu.VMEM((CH,), jnp.int32),
            pltpu.VMEM((CH, PW), _f32),
            pltpu.VMEM_SHARED((N_ACC, PW), _f32),
            pltpu.SemaphoreType.DMA,
        ],
    )(s1, s2, colsp, zsrc)


def _update_body(m0, m1, c0, c1, h, xp, u1a, u1b, bu1, u2, b2u, hn, xn):
    aggm = m0[0] + m1[0]
    aggc = c0[0] + c1[0]
    lane = lax.broadcasted_iota(jnp.int32, (1, PW), 1)
    mask3 = jnp.where(lane < 3, 1.0, 0.0).astype(_f32)
    deg = aggc[:, 3:4]
    xn[...] = xp[...] + (aggc * mask3) / (deg + 1.0)
    u = jax.nn.silu(h[...] @ u1a[...] + aggm @ u1b[...] + bu1[...])
    hn[...] = h[...] + u @ u2[...] + b2u[...]


def _update(aggm2, aggc2, h, xpad, u1a, u1b, bu1, u2, b2u):
    return pl.pallas_call(
        _update_body,
        grid=(NB,),
        in_specs=[
            pl.BlockSpec((1, BL, PW), lambda i: (0, i, 0)),
            pl.BlockSpec((1, BL, PW), lambda i: (1, i, 0)),
            pl.BlockSpec((1, BL, PW), lambda i: (0, i, 0)),
            pl.BlockSpec((1, BL, PW), lambda i: (1, i, 0)),
            pl.BlockSpec((BL, HID), lambda i: (i, 0)),
            pl.BlockSpec((BL, PW), lambda i: (i, 0)),
            _wspec((HID, HID)), _wspec((HID, HID)), _wspec((1, HID)),
            _wspec((HID, HID)), _wspec((1, HID)),
        ],
        out_specs=[
            pl.BlockSpec((BL, HID), lambda i: (i, 0)),
            pl.BlockSpec((BL, PW), lambda i: (i, 0)),
        ],
        out_shape=[
            jax.ShapeDtypeStruct((NN, HID), _f32),
            jax.ShapeDtypeStruct((NN, PW), _f32),
        ],
    )(aggm2, aggm2, aggc2, aggc2, h, xpad, u1a, u1b, bu1, u2, b2u)


def _final_body(h, xp, mk, wh, bh, vh, vx):
    vh[...] = (h[...] @ wh[...] + bh[...]) * mk[...]
    vx[...] = xp[...] * mk[...]


def _final(h, xpad, maskcol, wh, bh):
    return pl.pallas_call(
        _final_body,
        grid=(NB,),
        in_specs=[
            pl.BlockSpec((BL, HID), lambda i: (i, 0)),
            pl.BlockSpec((BL, PW), lambda i: (i, 0)),
            pl.BlockSpec((BL, 1), lambda i: (i, 0)),
            _wspec((HID, HID)), _wspec((1, HID)),
        ],
        out_specs=[
            pl.BlockSpec((BL, HID), lambda i: (i, 0)),
            pl.BlockSpec((BL, PW), lambda i: (i, 0)),
        ],
        out_shape=[
            jax.ShapeDtypeStruct((NN, HID), _f32),
            jax.ShapeDtypeStruct((NN, PW), _f32),
        ],
    )(h, xpad, maskcol, wh, bh)


@jax.jit
def _run(H_t, X_t, cond_embedding, edges, edge_types, generate_mask, t, params):
    row = edges[0].astype(jnp.int32)
    col = edges[1].astype(jnp.int32)
    et = edge_types.astype(jnp.int32)
    padsp = jnp.arange(E_PAD - EE, dtype=jnp.int32) % NN
    rowp = jnp.concatenate([row, padsp])
    colp = jnp.concatenate([col, padsp])
    etp = jnp.concatenate([et, jnp.zeros((E_PAD - EE,), jnp.int32)])
    colsp = jnp.concatenate(
        [col, NN + jnp.arange(E_PAD - EE, dtype=jnp.int32) % (N_ACC - NN)])
    xpad = jnp.pad(X_t, ((0, 0), (0, PW - 3)))
    tcol = t[:, None]
    maskcol = generate_mask.astype(_f32)[:, None]
    zsrc = jnp.zeros((N_ACC, PW), _f32)

    p0, p1, p2 = params["input_mlp"]
    h = _prelude(H_t, cond_embedding, tcol, p0, p1, p2)

    for lp in params["layers"]:
        w1 = lp["msg1"]["w"]
        b1 = lp["msg1"]["b"]
        teff = (params["edge_table"] @ w1[2 * HID + 1:] + b1[None, :])[:, None, :]
        wd2 = w1[2 * HID:2 * HID + 1]
        ta, tb2 = _tables(h, xpad, w1[:HID], w1[HID:2 * HID], teff)
        ga, gb = _sc_gather(ta, tb2, rowp, colp, etp)
        s1, s2 = _edge(ga, gb, wd2, lp["msg2"]["w"], lp["msg2"]["b"][None, :],
                       lp["coord"]["w"].T, lp["coord"]["b"][None, None, 0])
        aggm2, aggc2 = _sc_scatter(s1, s2, colsp, zsrc)
        u1 = lp["upd1"]["w"]
        h, xpad = _update(aggm2, aggc2, h, xpad, u1[:HID], u1[HID:],
                          lp["upd1"]["b"][None, :], lp["upd2"]["w"],
                          lp["upd2"]["b"][None, :])

    vh, vxp = _final(h, xpad, maskcol, params["hidden2input"]["w"],
                     params["hidden2input"]["b"][None, :])
    return vh, vxp[:, :3]


def kernel(H_t, X_t, cond_embedding, edges, edge_types, generate_mask,
           batch_ids, t, params):
    return _run(H_t, X_t, cond_embedding, edges, edge_types, generate_mask,
                t, params)


# speedup vs baseline: 3.2879x; 1.1370x over previous
import functools

import jax
import jax.numpy as jnp
from jax import lax
from jax.experimental import pallas as pl
from jax.experimental.pallas import tpu as pltpu
from jax.experimental.pallas import tpu_sc as plsc

NN = 10000
EE = 160000
HID = 128
GW = 256
PW = 128
NC, NS = 2, 16
NWORK = NC * NS
E_PAD = 163840
EPW = E_PAD // NWORK
CH = 128
NCHUNK = EPW // CH
N_ACC = 10112
RPS = N_ACC // NS
BL = 1000
NB = NN // BL
EBL = 1024
NEB = E_PAD // EBL

_f32 = jnp.float32


def _wspec(shape):
    return pl.BlockSpec(shape, lambda *_: tuple(0 for _ in shape))


def _prelude_body(ht, cond, tcol, w0a, w0b, w0c, b0, w1, b1, w2, b2, out):
    half = HID // 2
    i = lax.broadcasted_iota(jnp.int32, (1, half), 1).astype(_f32)
    freqs = jnp.exp(-jnp.log(10000.0) * i / (half - 1))
    ang = tcol[...] * freqs
    temb = jnp.concatenate([jnp.sin(ang), jnp.cos(ang)], axis=1)
    h = ht[...] @ w0a[...] + cond[...] @ w0b[...] + temb @ w0c[...] + b0[...]
    h = jax.nn.relu(h)
    h = jax.nn.relu(h @ w1[...] + b1[...])
    out[...] = h @ w2[...] + b2[...]


def _prelude(H_t, cond, tcol, p0, p1, p2):
    w0 = p0["w"]
    return pl.pallas_call(
        _prelude_body,
        grid=(NB,),
        in_specs=[
            pl.BlockSpec((BL, HID), lambda i: (i, 0)),
            pl.BlockSpec((BL, HID), lambda i: (i, 0)),
            pl.BlockSpec((BL, 1), lambda i: (i, 0)),
            _wspec((HID, HID)), _wspec((HID, HID)), _wspec((HID, HID)),
            _wspec((1, HID)), _wspec((HID, HID)), _wspec((1, HID)),
            _wspec((HID, HID)), _wspec((1, HID)),
        ],
        out_specs=pl.BlockSpec((BL, HID), lambda i: (i, 0)),
        out_shape=jax.ShapeDtypeStruct((NN, HID), _f32),
    )(H_t, cond, tcol, w0[:HID], w0[HID:2 * HID], w0[2 * HID:],
      p0["b"][None, :], p1["w"], p1["b"][None, :], p2["w"], p2["b"][None, :])


def _tables_body(h, xp, w1r, w1c, teff, ta, tb):
    a = jnp.concatenate([h[...] @ w1r[...], xp[...]], axis=1)
    ta[...] = a
    b = jnp.concatenate([h[...] @ w1c[...] + teff[0], xp[...]], axis=1)
    tb[...] = b


def _tables(h, xpad, w1r, w1c_b, teff):
    return pl.pallas_call(
        _tables_body,
        grid=(2, NB),
        in_specs=[
            pl.BlockSpec((BL, HID), lambda s, i: (i, 0)),
            pl.BlockSpec((BL, HID), lambda s, i: (i, 0)),
            _wspec((HID, HID)), _wspec((HID, HID)),
            pl.BlockSpec((1, 1, HID), lambda s, i: (s, 0, 0)),
        ],
        out_specs=[
            pl.BlockSpec((BL, GW), lambda s, i: (i, 0)),
            pl.BlockSpec((BL, GW), lambda s, i: (s * NB + i, 0)),
        ],
        out_shape=[
            jax.ShapeDtypeStruct((NN, GW), _f32),
            jax.ShapeDtypeStruct((2 * NN, GW), _f32),
        ],
    )(h, xpad, w1r, w1c_b, teff)


def _sc_gather_body(ta, tb2, rowp, colp, ga, gb,
                    idxr, idxc, bufa, bufb, sema, semb):
    wid = lax.axis_index("s") * NC + lax.axis_index("c")
    base0 = wid * EPW
    crow = wid * NCHUNK
    pltpu.sync_copy(rowp.at[pl.ds(crow, NCHUNK)], idxr)
    pltpu.sync_copy(colp.at[pl.ds(crow, NCHUNK)], idxc)

    def chunk(i, carry):
        base = base0 + i * CH
        ca = pltpu.async_copy(ta.at[idxr.at[i]], bufa, sema)
        cb = pltpu.async_copy(tb2.at[idxc.at[i]], bufb, semb)
        ca.wait()
        cb.wait()
        pltpu.sync_copy(bufa, ga.at[pl.ds(base, CH)])
        pltpu.sync_copy(bufb, gb.at[pl.ds(base, CH)])
        return carry

    lax.fori_loop(0, NCHUNK, chunk, 0)


def _sc_gather(ta, tb2, rowp, colp):
    mesh = plsc.VectorSubcoreMesh(core_axis_name="c", subcore_axis_name="s",
                                  num_cores=NC, num_subcores=NS)
    return pl.kernel(
        _sc_gather_body,
        out_type=[
            jax.ShapeDtypeStruct((E_PAD, GW), _f32),
            jax.ShapeDtypeStruct((E_PAD, GW), _f32),
        ],
        mesh=mesh,
        scratch_types=[
            pltpu.VMEM((NCHUNK, CH), jnp.int32),
            pltpu.VMEM((NCHUNK, CH), jnp.int32),
            pltpu.VMEM((CH, GW), _f32),
            pltpu.VMEM((CH, GW), _f32),
            pltpu.SemaphoreType.DMA,
            pltpu.SemaphoreType.DMA,
        ],
    )(ta, tb2, rowp, colp)


def _edge_body(ga, gb, wd2, w2, b2, wc, bc, om, oc):
    a = ga[...]
    b = gb[...]
    rel = a[:, HID:] - b[:, HID:]
    d2 = jnp.sum(rel * rel, axis=1, keepdims=True)
    z = a[:, :HID] + b[:, :HID] + d2 * wd2[...]
    m = jax.nn.silu(z)
    m = jax.nn.silu(m @ w2[...] + b2[...])
    wco = jnp.sum(m * wc[...], axis=1, keepdims=True) + bc[...]
    lane = lax.broadcasted_iota(jnp.int32, (1, PW), 1)
    one3 = jnp.where(lane == 3, 1.0, 0.0).astype(_f32)
    om[...] = m
    oc[...] = rel * wco + one3


def _edge(ga, gb, wd2, w2, b2, wc, bc):
    return pl.pallas_call(
        _edge_body,
        grid=(NEB,),
        in_specs=[
            pl.BlockSpec((EBL, GW), lambda i: (i, 0)),
            pl.BlockSpec((EBL, GW), lambda i: (i, 0)),
            _wspec((1, HID)), _wspec((HID, HID)), _wspec((1, HID)),
            _wspec((1, HID)), _wspec((1, 1)),
        ],
        out_specs=[
            pl.BlockSpec((EBL, PW), lambda i: (i, 0)),
            pl.BlockSpec((EBL, PW), lambda i: (i, 0)),
        ],
        out_shape=[
            jax.ShapeDtypeStruct((E_PAD, PW), _f32),
            jax.ShapeDtypeStruct((E_PAD, PW), _f32),
        ],
    )(ga, gb, wd2, w2, b2, wc, bc)


def _sc_scatter_body(s1, s2, colsp, zsrc, o1, o2, idxv, buf, shared, sem):
    c = lax.axis_index("c")
    s = lax.axis_index("s")
    wid = c * NS + s
    base0 = wid * EPW
    rsl = pl.ds(s * RPS, RPS)
    pltpu.sync_copy(colsp.at[pl.ds(wid * NCHUNK, NCHUNK)], idxv)

    for src, out in ((s1, o1), (s2, o2)):
        pltpu.sync_copy(zsrc.at[rsl], shared.at[rsl])
        plsc.subcore_barrier()

        def chunk(i, carry):
            base = base0 + i * CH
            pltpu.sync_copy(src.at[pl.ds(base, CH)], buf)
            pltpu.sync_copy(buf, shared.at[idxv.at[i]], add=True)
            return carry

        lax.fori_loop(0, NCHUNK, chunk, 0)
        plsc.subcore_barrier()
        pltpu.sync_copy(shared.at[rsl], out.at[c, rsl])
        plsc.subcore_barrier()


def _sc_scatter(s1, s2, colsp, zsrc):
    mesh = plsc.VectorSubcoreMesh(core_axis_name="c", subcore_axis_name="s",
                                  num_cores=NC, num_subcores=NS)
    return pl.kernel(
        _sc_scatter_body,
        out_type=[
            jax.ShapeDtypeStruct((NC, N_ACC, PW), _f32),
            jax.ShapeDtypeStruct((NC, N_ACC, PW), _f32),
        ],
        mesh=mesh,
        scratch_types=[
            pltpu.VMEM((NCHUNK, CH), jnp.int32),
            pltpu.VMEM((CH, PW), _f32),
            pltpu.VMEM_SHARED((N_ACC, PW), _f32),
            pltpu.SemaphoreType.DMA,
        ],
    )(s1, s2, colsp, zsrc)


def _update_body(m0, m1, c0, c1, h, xp, u1a, u1b, bu1, u2, b2u, hn, xn):
    aggm = m0[0] + m1[0]
    aggc = c0[0] + c1[0]
    lane = lax.broadcasted_iota(jnp.int32, (1, PW), 1)
    mask3 = jnp.where(lane < 3, 1.0, 0.0).astype(_f32)
    deg = aggc[:, 3:4]
    xn[...] = xp[...] + (aggc * mask3) / (deg + 1.0)
    u = jax.nn.silu(h[...] @ u1a[...] + aggm @ u1b[...] + bu1[...])
    hn[...] = h[...] + u @ u2[...] + b2u[...]


def _update(aggm2, aggc2, h, xpad, u1a, u1b, bu1, u2, b2u):
    return pl.pallas_call(
        _update_body,
        grid=(NB,),
        in_specs=[
            pl.BlockSpec((1, BL, PW), lambda i: (0, i, 0)),
            pl.BlockSpec((1, BL, PW), lambda i: (1, i, 0)),
            pl.BlockSpec((1, BL, PW), lambda i: (0, i, 0)),
            pl.BlockSpec((1, BL, PW), lambda i: (1, i, 0)),
            pl.BlockSpec((BL, HID), lambda i: (i, 0)),
            pl.BlockSpec((BL, PW), lambda i: (i, 0)),
            _wspec((HID, HID)), _wspec((HID, HID)), _wspec((1, HID)),
            _wspec((HID, HID)), _wspec((1, HID)),
        ],
        out_specs=[
            pl.BlockSpec((BL, HID), lambda i: (i, 0)),
            pl.BlockSpec((BL, PW), lambda i: (i, 0)),
        ],
        out_shape=[
            jax.ShapeDtypeStruct((NN, HID), _f32),
            jax.ShapeDtypeStruct((NN, PW), _f32),
        ],
    )(aggm2, aggm2, aggc2, aggc2, h, xpad, u1a, u1b, bu1, u2, b2u)


def _final_body(h, xp, mk, wh, bh, vh, vx):
    vh[...] = (h[...] @ wh[...] + bh[...]) * mk[...]
    vx[...] = xp[...] * mk[...]


def _final(h, xpad, maskcol, wh, bh):
    return pl.pallas_call(
        _final_body,
        grid=(NB,),
        in_specs=[
            pl.BlockSpec((BL, HID), lambda i: (i, 0)),
            pl.BlockSpec((BL, PW), lambda i: (i, 0)),
            pl.BlockSpec((BL, 1), lambda i: (i, 0)),
            _wspec((HID, HID)), _wspec((1, HID)),
        ],
        out_specs=[
            pl.BlockSpec((BL, HID), lambda i: (i, 0)),
            pl.BlockSpec((BL, PW), lambda i: (i, 0)),
        ],
        out_shape=[
            jax.ShapeDtypeStruct((NN, HID), _f32),
            jax.ShapeDtypeStruct((NN, PW), _f32),
        ],
    )(h, xpad, maskcol, wh, bh)


@jax.jit
def _run(H_t, X_t, cond_embedding, edges, edge_types, generate_mask, t, params):
    row = edges[0].astype(jnp.int32)
    col = edges[1].astype(jnp.int32)
    et = edge_types.astype(jnp.int32)
    padsp = jnp.arange(E_PAD - EE, dtype=jnp.int32) % NN
    rowp = jnp.concatenate([row, padsp]).reshape(E_PAD // CH, CH)
    colp = jnp.concatenate([col + et * NN, padsp]).reshape(E_PAD // CH, CH)
    colsp = jnp.concatenate(
        [col, NN + jnp.arange(E_PAD - EE, dtype=jnp.int32) % (N_ACC - NN)]
    ).reshape(E_PAD // CH, CH)
    xpad = jnp.pad(X_t, ((0, 0), (0, PW - 3)))
    tcol = t[:, None]
    maskcol = generate_mask.astype(_f32)[:, None]
    zsrc = jnp.zeros((N_ACC, PW), _f32)

    p0, p1, p2 = params["input_mlp"]
    h = _prelude(H_t, cond_embedding, tcol, p0, p1, p2)

    for lp in params["layers"]:
        w1 = lp["msg1"]["w"]
        b1 = lp["msg1"]["b"]
        teff = (params["edge_table"] @ w1[2 * HID + 1:] + b1[None, :])[:, None, :]
        wd2 = w1[2 * HID:2 * HID + 1]
        ta, tb2 = _tables(h, xpad, w1[:HID], w1[HID:2 * HID], teff)
        ga, gb = _sc_gather(ta, tb2, rowp, colp)
        s1, s2 = _edge(ga, gb, wd2, lp["msg2"]["w"], lp["msg2"]["b"][None, :],
                       lp["coord"]["w"].T, lp["coord"]["b"][None, None, 0])
        aggm2, aggc2 = _sc_scatter(s1, s2, colsp, zsrc)
        u1 = lp["upd1"]["w"]
        h, xpad = _update(aggm2, aggc2, h, xpad, u1[:HID], u1[HID:],
                          lp["upd1"]["b"][None, :], lp["upd2"]["w"],
                          lp["upd2"]["b"][None, :])

    vh, vxp = _final(h, xpad, maskcol, params["hidden2input"]["w"],
                     params["hidden2input"]["b"][None, :])
    return vh, vxp[:, :3]


def kernel(H_t, X_t, cond_embedding, edges, edge_types, generate_mask,
           batch_ids, t, params):
    return _run(H_t, X_t, cond_embedding, edges, edge_types, generate_mask,
                t, params)


# speedup vs baseline: 3.6956x; 1.1240x over previous
import functools

import jax
import jax.numpy as jnp
from jax import lax
from jax.experimental import pallas as pl
from jax.experimental.pallas import tpu as pltpu
from jax.experimental.pallas import tpu_sc as plsc

NN = 10000
EE = 160000
HID = 128
GW = 256
PW = 128
NC, NS = 2, 16
NWORK = NC * NS
E_PAD = 163840
EPW = E_PAD // NWORK
CH = 128
NCHUNK = EPW // CH
CHG = 64
NCHG = EPW // CHG
N_ACC = 10112
RPS = N_ACC // NS
BL = 1000
NB = NN // BL
EBL = 1024
NEB = E_PAD // EBL

_f32 = jnp.float32


def _wspec(shape):
    return pl.BlockSpec(shape, lambda *_: tuple(0 for _ in shape))


def _prelude_body(ht, cond, tcol, w0a, w0b, w0c, b0, w1, b1, w2, b2, out):
    half = HID // 2
    i = lax.broadcasted_iota(jnp.int32, (1, half), 1).astype(_f32)
    freqs = jnp.exp(-jnp.log(10000.0) * i / (half - 1))
    ang = tcol[...] * freqs
    temb = jnp.concatenate([jnp.sin(ang), jnp.cos(ang)], axis=1)
    h = ht[...] @ w0a[...] + cond[...] @ w0b[...] + temb @ w0c[...] + b0[...]
    h = jax.nn.relu(h)
    h = jax.nn.relu(h @ w1[...] + b1[...])
    out[...] = h @ w2[...] + b2[...]


def _prelude(H_t, cond, tcol, p0, p1, p2):
    w0 = p0["w"]
    return pl.pallas_call(
        _prelude_body,
        grid=(NB,),
        in_specs=[
            pl.BlockSpec((BL, HID), lambda i: (i, 0)),
            pl.BlockSpec((BL, HID), lambda i: (i, 0)),
            pl.BlockSpec((BL, 1), lambda i: (i, 0)),
            _wspec((HID, HID)), _wspec((HID, HID)), _wspec((HID, HID)),
            _wspec((1, HID)), _wspec((HID, HID)), _wspec((1, HID)),
            _wspec((HID, HID)), _wspec((1, HID)),
        ],
        out_specs=pl.BlockSpec((BL, HID), lambda i: (i, 0)),
        out_shape=jax.ShapeDtypeStruct((NN, HID), _f32),
    )(H_t, cond, tcol, w0[:HID], w0[HID:2 * HID], w0[2 * HID:],
      p0["b"][None, :], p1["w"], p1["b"][None, :], p2["w"], p2["b"][None, :])


def _tables_body(h, xp, w1r, w1c, teff, ta, tb):
    a = jnp.concatenate([h[...] @ w1r[...], xp[...]], axis=1)
    ta[...] = a
    b = jnp.concatenate([h[...] @ w1c[...] + teff[0], xp[...]], axis=1)
    tb[...] = b


def _tables(h, xpad, w1r, w1c_b, teff):
    return pl.pallas_call(
        _tables_body,
        grid=(2, NB),
        in_specs=[
            pl.BlockSpec((BL, HID), lambda s, i: (i, 0)),
            pl.BlockSpec((BL, HID), lambda s, i: (i, 0)),
            _wspec((HID, HID)), _wspec((HID, HID)),
            pl.BlockSpec((1, 1, HID), lambda s, i: (s, 0, 0)),
        ],
        out_specs=[
            pl.BlockSpec((BL, GW), lambda s, i: (i, 0)),
            pl.BlockSpec((BL, GW), lambda s, i: (s * NB + i, 0)),
        ],
        out_shape=[
            jax.ShapeDtypeStruct((NN, GW), _f32),
            jax.ShapeDtypeStruct((2 * NN, GW), _f32),
        ],
    )(h, xpad, w1r, w1c_b, teff)


def _sc_gather_body(ta, tb2, rowp, colp, ga, gb,
                    idxr, idxc, bufa, bufb, sema, semb):
    wid = lax.axis_index("s") * NC + lax.axis_index("c")
    base0 = wid * EPW
    crow = wid * NCHG
    pltpu.sync_copy(rowp.at[pl.ds(crow, NCHG)], idxr)
    pltpu.sync_copy(colp.at[pl.ds(crow, NCHG)], idxc)

    pltpu.async_copy(ta.at[idxr.at[0]], bufa.at[0], sema)
    pltpu.async_copy(tb2.at[idxc.at[0]], bufb.at[0], semb)

    def chunk(i, carry):
        base = base0 + i * CHG
        p = i % 2
        q = (i + 1) % 2
        nxt = jnp.minimum(i + 1, NCHG - 1)
        pltpu.async_copy(ta.at[idxr.at[nxt]], bufa.at[q], sema)
        pltpu.async_copy(tb2.at[idxc.at[nxt]], bufb.at[q], semb)
        pltpu.make_async_copy(ta.at[idxr.at[i]], bufa.at[p], sema).wait()
        pltpu.sync_copy(bufa.at[p], ga.at[pl.ds(base, CHG)])
        pltpu.make_async_copy(tb2.at[idxc.at[i]], bufb.at[p], semb).wait()
        pltpu.sync_copy(bufb.at[p], gb.at[pl.ds(base, CHG)])
        return carry

    lax.fori_loop(0, NCHG, chunk, 0)
    pltpu.make_async_copy(ta.at[idxr.at[0]], bufa.at[0], sema).wait()
    pltpu.make_async_copy(tb2.at[idxc.at[0]], bufb.at[0], semb).wait()


def _sc_gather(ta, tb2, rowp, colp):
    mesh = plsc.VectorSubcoreMesh(core_axis_name="c", subcore_axis_name="s",
                                  num_cores=NC, num_subcores=NS)
    return pl.kernel(
        _sc_gather_body,
        out_type=[
            jax.ShapeDtypeStruct((E_PAD, GW), _f32),
            jax.ShapeDtypeStruct((E_PAD, GW), _f32),
        ],
        mesh=mesh,
        scratch_types=[
            pltpu.VMEM((NCHG, CHG), jnp.int32),
            pltpu.VMEM((NCHG, CHG), jnp.int32),
            pltpu.VMEM((2, CHG, GW), _f32),
            pltpu.VMEM((2, CHG, GW), _f32),
            pltpu.SemaphoreType.DMA,
            pltpu.SemaphoreType.DMA,
        ],
    )(ta, tb2, rowp, colp)


def _edge_body(ga, gb, wd2, w2, b2, wc, bc, om, oc):
    a = ga[...]
    b = gb[...]
    rel = a[:, HID:] - b[:, HID:]
    d2 = jnp.sum(rel * rel, axis=1, keepdims=True)
    z = a[:, :HID] + b[:, :HID] + d2 * wd2[...]
    m = jax.nn.silu(z)
    m = jax.nn.silu(m @ w2[...] + b2[...])
    wco = jnp.sum(m * wc[...], axis=1, keepdims=True) + bc[...]
    lane = lax.broadcasted_iota(jnp.int32, (1, PW), 1)
    one3 = jnp.where(lane == 3, 1.0, 0.0).astype(_f32)
    om[...] = m
    oc[...] = rel * wco + one3


def _edge(ga, gb, wd2, w2, b2, wc, bc):
    return pl.pallas_call(
        _edge_body,
        grid=(NEB,),
        in_specs=[
            pl.BlockSpec((EBL, GW), lambda i: (i, 0)),
            pl.BlockSpec((EBL, GW), lambda i: (i, 0)),
            _wspec((1, HID)), _wspec((HID, HID)), _wspec((1, HID)),
            _wspec((1, HID)), _wspec((1, 1)),
        ],
        out_specs=[
            pl.BlockSpec((EBL, PW), lambda i: (i, 0)),
            pl.BlockSpec((EBL, PW), lambda i: (i, 0)),
        ],
        out_shape=[
            jax.ShapeDtypeStruct((E_PAD, PW), _f32),
            jax.ShapeDtypeStruct((E_PAD, PW), _f32),
        ],
    )(ga, gb, wd2, w2, b2, wc, bc)


def _sc_scatter_body(s1, s2, colsp, zsrc, o1, o2, idxv, buf, shared, sem):
    c = lax.axis_index("c")
    s = lax.axis_index("s")
    wid = c * NS + s
    base0 = wid * EPW
    rsl = pl.ds(s * RPS, RPS)
    pltpu.sync_copy(colsp.at[pl.ds(wid * NCHUNK, NCHUNK)], idxv)

    for src, out in ((s1, o1), (s2, o2)):
        pltpu.sync_copy(zsrc.at[rsl], shared.at[rsl])
        plsc.subcore_barrier()
        pltpu.async_copy(src.at[pl.ds(base0, CH)], buf.at[0], sem)

        def chunk(i, carry):
            p = i % 2
            q = (i + 1) % 2
            nxt = jnp.minimum(i + 1, NCHUNK - 1)
            pltpu.async_copy(src.at[pl.ds(base0 + nxt * CH, CH)],
                             buf.at[q], sem)
            pltpu.make_async_copy(src.at[pl.ds(base0 + i * CH, CH)],
                                  buf.at[p], sem).wait()
            pltpu.sync_copy(buf.at[p], shared.at[idxv.at[i]], add=True)
            return carry

        lax.fori_loop(0, NCHUNK, chunk, 0)
        pltpu.make_async_copy(src.at[pl.ds(base0, CH)], buf.at[0], sem).wait()
        plsc.subcore_barrier()
        pltpu.sync_copy(shared.at[rsl], out.at[c, rsl])
        plsc.subcore_barrier()


def _sc_scatter(s1, s2, colsp, zsrc):
    mesh = plsc.VectorSubcoreMesh(core_axis_name="c", subcore_axis_name="s",
                                  num_cores=NC, num_subcores=NS)
    return pl.kernel(
        _sc_scatter_body,
        out_type=[
            jax.ShapeDtypeStruct((NC, N_ACC, PW), _f32),
            jax.ShapeDtypeStruct((NC, N_ACC, PW), _f32),
        ],
        mesh=mesh,
        scratch_types=[
            pltpu.VMEM((NCHUNK, CH), jnp.int32),
            pltpu.VMEM((2, CH, PW), _f32),
            pltpu.VMEM_SHARED((N_ACC, PW), _f32),
            pltpu.SemaphoreType.DMA,
        ],
    )(s1, s2, colsp, zsrc)


def _update_body(m0, m1, c0, c1, h, xp, u1a, u1b, bu1, u2, b2u, hn, xn):
    aggm = m0[0] + m1[0]
    aggc = c0[0] + c1[0]
    lane = lax.broadcasted_iota(jnp.int32, (1, PW), 1)
    mask3 = jnp.where(lane < 3, 1.0, 0.0).astype(_f32)
    deg = aggc[:, 3:4]
    xn[...] = xp[...] + (aggc * mask3) / (deg + 1.0)
    u = jax.nn.silu(h[...] @ u1a[...] + aggm @ u1b[...] + bu1[...])
    hn[...] = h[...] + u @ u2[...] + b2u[...]


def _update(aggm2, aggc2, h, xpad, u1a, u1b, bu1, u2, b2u):
    return pl.pallas_call(
        _update_body,
        grid=(NB,),
        in_specs=[
            pl.BlockSpec((1, BL, PW), lambda i: (0, i, 0)),
            pl.BlockSpec((1, BL, PW), lambda i: (1, i, 0)),
            pl.BlockSpec((1, BL, PW), lambda i: (0, i, 0)),
            pl.BlockSpec((1, BL, PW), lambda i: (1, i, 0)),
            pl.BlockSpec((BL, HID), lambda i: (i, 0)),
            pl.BlockSpec((BL, PW), lambda i: (i, 0)),
            _wspec((HID, HID)), _wspec((HID, HID)), _wspec((1, HID)),
            _wspec((HID, HID)), _wspec((1, HID)),
        ],
        out_specs=[
            pl.BlockSpec((BL, HID), lambda i: (i, 0)),
            pl.BlockSpec((BL, PW), lambda i: (i, 0)),
        ],
        out_shape=[
            jax.ShapeDtypeStruct((NN, HID), _f32),
            jax.ShapeDtypeStruct((NN, PW), _f32),
        ],
    )(aggm2, aggm2, aggc2, aggc2, h, xpad, u1a, u1b, bu1, u2, b2u)


def _final_body(h, xp, mk, wh, bh, vh, vx):
    vh[...] = (h[...] @ wh[...] + bh[...]) * mk[...]
    vx[...] = xp[...] * mk[...]


def _final(h, xpad, maskcol, wh, bh):
    return pl.pallas_call(
        _final_body,
        grid=(NB,),
        in_specs=[
            pl.BlockSpec((BL, HID), lambda i: (i, 0)),
            pl.BlockSpec((BL, PW), lambda i: (i, 0)),
            pl.BlockSpec((BL, 1), lambda i: (i, 0)),
            _wspec((HID, HID)), _wspec((1, HID)),
        ],
        out_specs=[
            pl.BlockSpec((BL, HID), lambda i: (i, 0)),
            pl.BlockSpec((BL, PW), lambda i: (i, 0)),
        ],
        out_shape=[
            jax.ShapeDtypeStruct((NN, HID), _f32),
            jax.ShapeDtypeStruct((NN, PW), _f32),
        ],
    )(h, xpad, maskcol, wh, bh)


@jax.jit
def _run(H_t, X_t, cond_embedding, edges, edge_types, generate_mask, t, params):
    row = edges[0].astype(jnp.int32)
    col = edges[1].astype(jnp.int32)
    et = edge_types.astype(jnp.int32)
    padsp = jnp.arange(E_PAD - EE, dtype=jnp.int32) % NN
    rowp = jnp.concatenate([row, padsp]).reshape(E_PAD // CHG, CHG)
    colp = jnp.concatenate([col + et * NN, padsp]).reshape(E_PAD // CHG, CHG)
    colsp = jnp.concatenate(
        [col, NN + jnp.arange(E_PAD - EE, dtype=jnp.int32) % (N_ACC - NN)]
    ).reshape(E_PAD // CH, CH)
    xpad = jnp.pad(X_t, ((0, 0), (0, PW - 3)))
    tcol = t[:, None]
    maskcol = generate_mask.astype(_f32)[:, None]
    zsrc = jnp.zeros((N_ACC, PW), _f32)

    p0, p1, p2 = params["input_mlp"]
    h = _prelude(H_t, cond_embedding, tcol, p0, p1, p2)

    for lp in params["layers"]:
        w1 = lp["msg1"]["w"]
        b1 = lp["msg1"]["b"]
        teff = (params["edge_table"] @ w1[2 * HID + 1:] + b1[None, :])[:, None, :]
        wd2 = w1[2 * HID:2 * HID + 1]
        ta, tb2 = _tables(h, xpad, w1[:HID], w1[HID:2 * HID], teff)
        ga, gb = _sc_gather(ta, tb2, rowp, colp)
        s1, s2 = _edge(ga, gb, wd2, lp["msg2"]["w"], lp["msg2"]["b"][None, :],
                       lp["coord"]["w"].T, lp["coord"]["b"][None, None, 0])
        aggm2, aggc2 = _sc_scatter(s1, s2, colsp, zsrc)
        u1 = lp["upd1"]["w"]
        h, xpad = _update(aggm2, aggc2, h, xpad, u1[:HID], u1[HID:],
                          lp["upd1"]["b"][None, :], lp["upd2"]["w"],
                          lp["upd2"]["b"][None, :])

    vh, vxp = _final(h, xpad, maskcol, params["hidden2input"]["w"],
                     params["hidden2input"]["b"][None, :])
    return vh, vxp[:, :3]


def kernel(H_t, X_t, cond_embedding, edges, edge_types, generate_mask,
           batch_ids, t, params):
    return _run(H_t, X_t, cond_embedding, edges, edge_types, generate_mask,
                t, params)
